# baseline (speedup 1.0000x reference)
import functools

import jax
import jax.numpy as jnp
from jax import lax
from jax.experimental import pallas as pl
from jax.experimental.pallas import tpu as pltpu
from jax.experimental.pallas import tpu_sc as plsc

VOCAB = 1000000
DIM = 64
BATCH = 16384

_INFO = plsc.get_sparse_core_info()
NC = _INFO.num_cores
NS = _INFO.num_subcores
LANES = _INFO.num_lanes
NW = NC * NS
BPW = BATCH // NW
CH = 128
NPH = BPW // CH
GPP = CH // LANES

_mesh = plsc.VectorSubcoreMesh(core_axis_name="c", subcore_axis_name="s")


@functools.partial(
    pl.kernel,
    mesh=_mesh,
    compiler_params=pltpu.CompilerParams(needs_layout_passes=False,
                                         use_tc_tiling_on_sc=False),
    out_type=jax.ShapeDtypeStruct((BATCH,), jnp.float32),
    scratch_types=[
        pltpu.VMEM((BPW,), jnp.int32),
        pltpu.VMEM((BPW,), jnp.int32),
        pltpu.VMEM((BPW,), jnp.int32),
        pltpu.VMEM((BPW,), jnp.int32),
        pltpu.VMEM((BPW,), jnp.int32),
        pltpu.VMEM((BPW,), jnp.int32),
        pltpu.VMEM((2, CH, 2 * DIM), jnp.float32),
        pltpu.VMEM((2, CH, 2 * DIM), jnp.float32),
        pltpu.VMEM((BPW,), jnp.float32),
        pltpu.VMEM((BPW,), jnp.float32),
        pltpu.VMEM((BPW,), jnp.float32),
        pltpu.SemaphoreType.DMA,
        pltpu.SemaphoreType.DMA,
        pltpu.SemaphoreType.DMA,
    ],
)
def _glove_sc(cid_hbm, xid_hbm, cidh_hbm, xidh_hbm, cidl_hbm, xidl_hbm,
              wemb_hbm, wctx_hbm, bw_hbm, bc_hbm, out_hbm,
              cid_v, xid_v, cidh_v, xidh_v, cidl_v, xidl_v, wrows, crows,
              bw_f, bc_f, out_v, sem0, sem1, bsem):
    wid = lax.axis_index("s") * NC + lax.axis_index("c")
    base = wid * BPW

    pltpu.sync_copy(cid_hbm.at[pl.ds(base, BPW)], cid_v)
    pltpu.sync_copy(xid_hbm.at[pl.ds(base, BPW)], xid_v)
    pltpu.sync_copy(cidh_hbm.at[pl.ds(base, BPW)], cidh_v)
    pltpu.sync_copy(xidh_hbm.at[pl.ds(base, BPW)], xidh_v)
    pltpu.sync_copy(cidl_hbm.at[pl.ds(base, BPW)], cidl_v)
    pltpu.sync_copy(xidl_hbm.at[pl.ds(base, BPW)], xidl_v)

    bias_copies = []
    for j in range(NPH):
        sl = pl.ds(j * CH, CH)
        bias_copies.append(pltpu.async_copy(bw_hbm.at[cid_v.at[sl]],
                                            bw_f.at[sl], bsem))
        bias_copies.append(pltpu.async_copy(bc_hbm.at[xid_v.at[sl]],
                                            bc_f.at[sl], bsem))

    sems = [sem0, sem1]

    def fire(p):
        sl = pl.ds(p * CH, CH)
        buf = p % 2
        s = sems[buf]
        return (pltpu.async_copy(wemb_hbm.at[cidh_v.at[sl]],
                                 wrows.at[buf], s),
                pltpu.async_copy(wctx_hbm.at[xidh_v.at[sl]],
                                 crows.at[buf], s))

    inflight = {0: fire(0)}
    iota = lax.iota(jnp.int32, LANES)

    for cp in bias_copies:
        cp.wait()

    for p in range(NPH):
        if p + 1 < NPH:
            inflight[p + 1] = fire(p + 1)
        for cp in inflight.pop(p):
            cp.wait()
        buf = p % 2
        wb = wrows.at[buf]
        cb = crows.at[buf]

        def group(g, carry, p=p, wb=wb, cb=cb):
            b0 = p * CH + g * LANES
            row = g * LANES + iota
            wcol = cidl_v[pl.ds(b0, LANES)] * DIM
            ccol = xidl_v[pl.ds(b0, LANES)] * DIM
            acc0 = bw_f[pl.ds(b0, LANES)] + bc_f[pl.ds(b0, LANES)]
            acc1 = jnp.zeros((LANES,), jnp.float32)
            acc2 = jnp.zeros((LANES,), jnp.float32)
            acc3 = jnp.zeros((LANES,), jnp.float32)
            accs = [acc0, acc1, acc2, acc3]
            one = jnp.ones((LANES,), jnp.int32)
            for d in range(DIM):
                w = plsc.load_gather(wb, [row, wcol])
                c = plsc.load_gather(cb, [row, ccol])
                accs[d % 4] = accs[d % 4] + w * c
                wcol = wcol + one
                ccol = ccol + one
            out_v[pl.ds(b0, LANES)] = (accs[0] + accs[1]) + (accs[2] + accs[3])
            return carry

        lax.fori_loop(0, GPP, group, 0)

    pltpu.sync_copy(out_v, out_hbm.at[pl.ds(base, BPW)])


def kernel(center_ids, context_ids, W_emb, W_ctx, b_w, b_c):
    cid = center_ids.astype(jnp.int32)
    xid = context_ids.astype(jnp.int32)
    w2, c2 = lax.optimization_barrier(
        (W_emb.reshape(VOCAB // 2, 2 * DIM),
         W_ctx.reshape(VOCAB // 2, 2 * DIM)))
    return _glove_sc(cid, xid, cid >> 1, xid >> 1, cid & 1, xid & 1,
                     w2, c2, b_w.reshape(VOCAB), b_c.reshape(VOCAB))
